# Initial kernel scaffold; baseline (speedup 1.0000x reference)
import jax
import jax.numpy as jnp
from jax.experimental import pallas as pl


def kernel(x, weight):
    raise NotImplementedError("write your pallas kernel here")


# speedup vs baseline: 4.2556x; 4.2556x over previous
import functools

import jax
import jax.numpy as jnp
from jax.experimental import pallas as pl
from jax.experimental.pallas import tpu as pltpu
from jax.experimental.pallas import tpu_sc as plsc

_WINDOW = 128


@functools.partial(jax.jit, static_argnames=())
def _gather_rows(weight, idx_flat):
    n = idx_flat.shape[0]
    d = weight.shape[1]
    idx2d = idx_flat.reshape(1, n)

    mesh = plsc.VectorSubcoreMesh(
        core_axis_name="core", subcore_axis_name="subcore"
    )

    @functools.partial(
        pl.kernel,
        out_type=jax.ShapeDtypeStruct((n, d), weight.dtype),
        mesh=mesh,
        compiler_params=pltpu.CompilerParams(use_tc_tiling_on_sc=False),
    )
    def k(w_hbm, i_hbm, o_hbm):
        def body(i_vmem, o_vmem):
            pltpu.sync_copy(w_hbm.at[i_vmem.at[0]], o_vmem)

        pltpu.emit_pipeline(
            body,
            grid=(n // _WINDOW,),
            in_specs=[pl.BlockSpec((1, _WINDOW), index_map=lambda i: (0, i))],
            out_specs=[pl.BlockSpec((_WINDOW, d), index_map=lambda i: (i, 0))],
            core_axis_name=("core", "subcore"),
            dimension_semantics=(pltpu.PARALLEL,),
        )(i_hbm, o_hbm)

    return k(weight, idx2d)


def kernel(x, weight):
    b, h = x.shape
    out = _gather_rows(weight, x.reshape(b * h).astype(jnp.int32))
    return out.reshape(b, h, weight.shape[1])


# speedup vs baseline: 4.6629x; 1.0957x over previous
import functools

import jax
import jax.numpy as jnp
from jax import lax
from jax.experimental import pallas as pl
from jax.experimental.pallas import tpu as pltpu
from jax.experimental.pallas import tpu_sc as plsc

_NW = 32
_CHUNK = 640
_GATHER = 640


def _gather_rows(weight, idx_flat):
    n = idx_flat.shape[0]
    d = weight.shape[1]
    b_per_w = n // _NW
    nchunk = b_per_w // _CHUNK
    ng = _CHUNK // _GATHER

    mesh = plsc.VectorSubcoreMesh(core_axis_name="c", subcore_axis_name="s")

    @functools.partial(
        pl.kernel,
        out_type=jax.ShapeDtypeStruct((n, d), weight.dtype),
        mesh=mesh,
        compiler_params=pltpu.CompilerParams(use_tc_tiling_on_sc=False),
        scratch_types=[
            pltpu.VMEM((b_per_w,), jnp.int32),
            pltpu.VMEM((2, _CHUNK, d), jnp.float32),
            pltpu.SemaphoreType.DMA,
            pltpu.SemaphoreType.DMA,
            pltpu.SemaphoreType.DMA,
            pltpu.SemaphoreType.DMA,
        ],
    )
    def k(w_hbm, i_hbm, o_hbm, idx_v, rows_v, g0, g1, o0, o1):
        gsem = (g0, g1)
        osem = (o0, o1)
        wid = lax.axis_index("s") * 2 + lax.axis_index("c")
        base = wid * b_per_w
        pltpu.sync_copy(i_hbm.at[pl.ds(base, b_per_w)], idx_v)

        gh = [[] for _ in range(2)]
        oh = [None, None]

        def fire_gathers(c):
            buf = c % 2
            for j in range(ng):
                gh[buf].append(
                    pltpu.async_copy(
                        w_hbm.at[idx_v.at[pl.ds(c * _CHUNK + j * _GATHER, _GATHER)]],
                        rows_v.at[buf, pl.ds(j * _GATHER, _GATHER)],
                        gsem[buf],
                    )
                )

        fire_gathers(0)
        for c in range(nchunk):
            buf = c % 2
            if c + 1 < nchunk:
                nbuf = (c + 1) % 2
                if oh[nbuf] is not None:
                    oh[nbuf].wait()
                    oh[nbuf] = None
                fire_gathers(c + 1)
            for hdl in gh[buf]:
                hdl.wait()
            gh[buf] = []
            oh[buf] = pltpu.async_copy(
                rows_v.at[buf], o_hbm.at[pl.ds(base + c * _CHUNK, _CHUNK)], osem[buf]
            )
        for buf in range(2):
            if oh[buf] is not None:
                oh[buf].wait()

    return k(weight, idx_flat)


def kernel(x, weight):
    b, h = x.shape
    out = _gather_rows(weight, x.reshape(b * h).astype(jnp.int32))
    return out.reshape(b, h, weight.shape[1])
